# Initial kernel scaffold; baseline (speedup 1.0000x reference)
import jax
import jax.numpy as jnp
from jax.experimental import pallas as pl


def kernel(fine_ids, coarse_ids, fine_W, coarse_W, W1, b1, W2, b2):
    raise NotImplementedError("write your pallas kernel here")


# speedup vs baseline: 2.6008x; 2.6008x over previous
import functools

import jax
import jax.numpy as jnp
from jax import lax
from jax.experimental import pallas as pl
from jax.experimental.pallas import tpu as pltpu
from jax.experimental.pallas import tpu_sc as plsc

CHUNK = 128


def _sc_gather_body(nchunks, dim, fi_hbm, ci_hbm, fw_hbm, cw_hbm,
                    fout_hbm, cout_hbm, idx_v, rows_v, sem):
    nc = 2
    wid = lax.axis_index("s") * nc + lax.axis_index("c")
    base = wid * nchunks * CHUNK

    def do_table(ids_hbm, table_hbm, out_hbm):
        pltpu.sync_copy(ids_hbm.at[wid], idx_v)

        def body(j, carry):
            pltpu.async_copy(table_hbm.at[idx_v.at[j]], rows_v, sem).wait()
            pltpu.sync_copy(rows_v, out_hbm.at[pl.ds(base + j * CHUNK, CHUNK)])
            return carry

        lax.fori_loop(0, nchunks, body, 0)

    do_table(fi_hbm, fw_hbm, fout_hbm)
    do_table(ci_hbm, cw_hbm, cout_hbm)


def _make_sc_gather(n_rows, dim, nw):
    nchunks = n_rows // (nw * CHUNK)
    mesh = plsc.VectorSubcoreMesh(core_axis_name="c", subcore_axis_name="s")
    return functools.partial(
        pl.kernel,
        mesh=mesh,
        compiler_params=pltpu.CompilerParams(use_tc_tiling_on_sc=False),
        out_type=[
            jax.ShapeDtypeStruct((n_rows, dim), jnp.float32),
            jax.ShapeDtypeStruct((n_rows, dim), jnp.float32),
        ],
        scratch_types=[
            pltpu.VMEM((nchunks, CHUNK), jnp.int32),
            pltpu.VMEM((CHUNK, dim), jnp.float32),
            pltpu.SemaphoreType.DMA,
        ],
    )(functools.partial(_sc_gather_body, nchunks, dim))


def _mlp_body(fine_ref, coarse_ref, w1a_ref, w1b_ref, b1_ref, w2_ref, b2_ref,
              out_ref, g_ref):
    f = fine_ref[...]
    c = coarse_ref[...]
    h = jnp.dot(f, w1a_ref[...], preferred_element_type=jnp.float32)
    h = h + jnp.dot(c, w1b_ref[...], preferred_element_type=jnp.float32)
    h = jnp.maximum(h + b1_ref[...], 0.0)
    z = jnp.sum(h * w2_ref[...], axis=1, keepdims=True) + b2_ref[...]
    g = jax.nn.sigmoid(z)
    out_ref[...] = g * f + (1.0 - g) * c
    g_ref[...] = g


def _tc_mlp(fine_rows, coarse_rows, W1, b1, W2, b2, rblk=4096):
    n, dim = fine_rows.shape
    w1a = W1[:dim, :]
    w1b = W1[dim:, :]
    b1r = b1.reshape(1, -1)
    w2r = W2.reshape(1, -1)
    b2r = b2.reshape(1, 1)
    hdim = W1.shape[1]
    grid = (n // rblk,)
    return pl.pallas_call(
        _mlp_body,
        grid=grid,
        in_specs=[
            pl.BlockSpec((rblk, dim), lambda i: (i, 0)),
            pl.BlockSpec((rblk, dim), lambda i: (i, 0)),
            pl.BlockSpec((dim, hdim), lambda i: (0, 0)),
            pl.BlockSpec((dim, hdim), lambda i: (0, 0)),
            pl.BlockSpec((1, hdim), lambda i: (0, 0)),
            pl.BlockSpec((1, hdim), lambda i: (0, 0)),
            pl.BlockSpec((1, 1), lambda i: (0, 0)),
        ],
        out_specs=[
            pl.BlockSpec((rblk, dim), lambda i: (i, 0)),
            pl.BlockSpec((rblk, 1), lambda i: (i, 0)),
        ],
        out_shape=[
            jax.ShapeDtypeStruct((n, dim), jnp.float32),
            jax.ShapeDtypeStruct((n, 1), jnp.float32),
        ],
    )(fine_rows, coarse_rows, w1a, w1b, b1r, w2r, b2r)


def kernel(fine_ids, coarse_ids, fine_W, coarse_W, W1, b1, W2, b2):
    b, l = fine_ids.shape
    dim = fine_W.shape[1]
    n = b * l
    nw = 32
    nchunks = n // (nw * CHUNK)
    fi = fine_ids.astype(jnp.int32).reshape(nw, nchunks, CHUNK)
    ci = coarse_ids.astype(jnp.int32).reshape(nw, nchunks, CHUNK)
    fine_rows, coarse_rows = _make_sc_gather(n, dim, nw)(
        fi, ci, fine_W, coarse_W)
    out_flat, g_flat = _tc_mlp(fine_rows, coarse_rows, W1, b1, W2, b2)
    return out_flat.reshape(b, l, dim), g_flat.reshape(b, l, 1)


# speedup vs baseline: 3.4427x; 1.3237x over previous
import functools

import jax
import jax.numpy as jnp
from jax import lax
from jax.experimental import pallas as pl
from jax.experimental.pallas import tpu as pltpu
from jax.experimental.pallas import tpu_sc as plsc

CHUNK = 128


def _sc_gather_body(nchunks, dim, fi_hbm, ci_hbm, fw_hbm, cw_hbm,
                    cat_hbm, idx_f, idx_c, rows_f, rows_c, sem_f, sem_c):
    nc = 2
    wid = lax.axis_index("s") * nc + lax.axis_index("c")
    base = wid * nchunks * CHUNK

    pltpu.sync_copy(fi_hbm.at[wid], idx_f)
    pltpu.sync_copy(ci_hbm.at[wid], idx_c)

    def body(j, carry):
        r = base + j * CHUNK
        cf = pltpu.async_copy(fw_hbm.at[idx_f.at[j]], rows_f, sem_f)
        cc = pltpu.async_copy(cw_hbm.at[idx_c.at[j]], rows_c, sem_c)
        cf.wait()
        cc.wait()
        pltpu.sync_copy(rows_f, cat_hbm.at[pl.ds(r, CHUNK), pl.ds(0, dim)])
        pltpu.sync_copy(rows_c, cat_hbm.at[pl.ds(r, CHUNK), pl.ds(dim, dim)])
        return carry

    lax.fori_loop(0, nchunks, body, 0)


def _make_sc_gather(n_rows, dim, nw):
    nchunks = n_rows // (nw * CHUNK)
    mesh = plsc.VectorSubcoreMesh(core_axis_name="c", subcore_axis_name="s")
    return functools.partial(
        pl.kernel,
        mesh=mesh,
        compiler_params=pltpu.CompilerParams(use_tc_tiling_on_sc=False),
        out_type=jax.ShapeDtypeStruct((n_rows, 2 * dim), jnp.float32),
        scratch_types=[
            pltpu.VMEM((nchunks, CHUNK), jnp.int32),
            pltpu.VMEM((nchunks, CHUNK), jnp.int32),
            pltpu.VMEM((CHUNK, dim), jnp.float32),
            pltpu.VMEM((CHUNK, dim), jnp.float32),
            pltpu.SemaphoreType.DMA,
            pltpu.SemaphoreType.DMA,
        ],
    )(functools.partial(_sc_gather_body, nchunks, dim))


def _mlp_body(dim, cat_ref, w1_ref, b1_ref, w2_ref, b2_ref, out_ref, g_ref):
    cat = cat_ref[...]
    h = jnp.dot(cat, w1_ref[...], preferred_element_type=jnp.float32)
    h = jnp.maximum(h + b1_ref[...], 0.0)
    z = jnp.sum(h * w2_ref[...], axis=1, keepdims=True) + b2_ref[...]
    g = jax.nn.sigmoid(z)
    f = cat[:, :dim]
    c = cat[:, dim:]
    out_ref[...] = c + g * (f - c)
    g_ref[...] = g[:, 0]


def _tc_mlp(cat, W1, b1, W2, b2, rblk=4096):
    n, two_dim = cat.shape
    dim = two_dim // 2
    b1r = b1.reshape(1, -1)
    w2r = W2.reshape(1, -1)
    b2r = b2.reshape(1, 1)
    hdim = W1.shape[1]
    grid = (n // rblk,)
    return pl.pallas_call(
        functools.partial(_mlp_body, dim),
        grid=grid,
        in_specs=[
            pl.BlockSpec((rblk, two_dim), lambda i: (i, 0)),
            pl.BlockSpec((two_dim, hdim), lambda i: (0, 0)),
            pl.BlockSpec((1, hdim), lambda i: (0, 0)),
            pl.BlockSpec((1, hdim), lambda i: (0, 0)),
            pl.BlockSpec((1, 1), lambda i: (0, 0)),
        ],
        out_specs=[
            pl.BlockSpec((rblk, dim), lambda i: (i, 0)),
            pl.BlockSpec((rblk,), lambda i: (i,)),
        ],
        out_shape=[
            jax.ShapeDtypeStruct((n, dim), jnp.float32),
            jax.ShapeDtypeStruct((n,), jnp.float32),
        ],
    )(cat, W1, b1r, w2r, b2r)


def kernel(fine_ids, coarse_ids, fine_W, coarse_W, W1, b1, W2, b2):
    b, l = fine_ids.shape
    dim = fine_W.shape[1]
    n = b * l
    nw = 32
    nchunks = n // (nw * CHUNK)
    fi = fine_ids.astype(jnp.int32).reshape(nw, nchunks, CHUNK)
    ci = coarse_ids.astype(jnp.int32).reshape(nw, nchunks, CHUNK)
    cat = _make_sc_gather(n, dim, nw)(fi, ci, fine_W, coarse_W)
    out_flat, g_flat = _tc_mlp(cat, W1, b1, W2, b2)
    return out_flat.reshape(b, l, dim), g_flat.reshape(b, l, 1)


# speedup vs baseline: 6.0277x; 1.7509x over previous
import functools

import jax
import jax.numpy as jnp
from jax import lax
from jax.experimental import pallas as pl
from jax.experimental.pallas import tpu as pltpu
from jax.experimental.pallas import tpu_sc as plsc

CHUNK = 128


def _sc_gather_body(nchunks, dim, fi_hbm, ci_hbm, fw_hbm, cw_hbm,
                    cat_hbm, idx_f, idx_c, rows_f, rows_c, sem_f, sem_c):
    nc = 2
    wid = lax.axis_index("s") * nc + lax.axis_index("c")
    base = wid * nchunks * CHUNK

    pltpu.sync_copy(fi_hbm.at[wid], idx_f)
    pltpu.sync_copy(ci_hbm.at[wid], idx_c)

    def body(j, carry):
        r = base + j * CHUNK
        cf = pltpu.async_copy(fw_hbm.at[idx_f.at[j]], rows_f, sem_f)
        cc = pltpu.async_copy(cw_hbm.at[idx_c.at[j]], rows_c, sem_c)
        cf.wait()
        cc.wait()
        pltpu.sync_copy(rows_f, cat_hbm.at[pl.ds(r, CHUNK), pl.ds(0, dim)])
        pltpu.sync_copy(rows_c, cat_hbm.at[pl.ds(r, CHUNK), pl.ds(dim, dim)])
        return carry

    lax.fori_loop(0, nchunks, body, 0)


def _make_sc_gather(n_rows, dim, nw):
    nchunks = n_rows // (nw * CHUNK)
    mesh = plsc.VectorSubcoreMesh(core_axis_name="c", subcore_axis_name="s")
    return functools.partial(
        pl.kernel,
        mesh=mesh,
        compiler_params=pltpu.CompilerParams(use_tc_tiling_on_sc=False),
        out_type=jax.ShapeDtypeStruct((n_rows, 2 * dim), jnp.float32),
        scratch_types=[
            pltpu.VMEM((nchunks, CHUNK), jnp.int32),
            pltpu.VMEM((nchunks, CHUNK), jnp.int32),
            pltpu.VMEM((CHUNK, dim), jnp.float32),
            pltpu.VMEM((CHUNK, dim), jnp.float32),
            pltpu.SemaphoreType.DMA,
            pltpu.SemaphoreType.DMA,
        ],
    )(functools.partial(_sc_gather_body, nchunks, dim))


def _mlp_body(dim, cat_ref, w1_ref, b1_ref, w2_ref, b2_ref, out_ref, g_ref):
    cat_t = cat_ref[...].T
    h = lax.dot_general(w1_ref[...], cat_t, (((0,), (0,)), ((), ())),
                        preferred_element_type=jnp.float32)
    h = jnp.maximum(h + b1_ref[...], 0.0)
    z = jnp.sum(h * w2_ref[...], axis=0, keepdims=True) + b2_ref[...]
    g = jax.nn.sigmoid(z)
    f = cat_t[:dim, :]
    c = cat_t[dim:, :]
    out_ref[0] = c + g * (f - c)
    g_ref[...] = g[None]


def _tc_mlp(cat, W1, b1, W2, b2, b, l, rblk=4096):
    n, two_dim = cat.shape
    dim = two_dim // 2
    b1c = b1.reshape(-1, 1)
    b2c = b2.reshape(1, 1)
    hdim = W1.shape[1]
    bpl = b // rblk
    grid = (n // rblk,)
    return pl.pallas_call(
        functools.partial(_mlp_body, dim),
        grid=grid,
        in_specs=[
            pl.BlockSpec((rblk, two_dim), lambda i: (i, 0)),
            pl.BlockSpec((two_dim, hdim), lambda i: (0, 0)),
            pl.BlockSpec((hdim, 1), lambda i: (0, 0)),
            pl.BlockSpec((hdim, 1), lambda i: (0, 0)),
            pl.BlockSpec((1, 1), lambda i: (0, 0)),
        ],
        out_specs=[
            pl.BlockSpec((1, dim, rblk), lambda i: (i // bpl, 0, i % bpl)),
            pl.BlockSpec((1, 1, rblk), lambda i: (i // bpl, 0, i % bpl)),
        ],
        out_shape=[
            jax.ShapeDtypeStruct((l, dim, b), jnp.float32),
            jax.ShapeDtypeStruct((l, 1, b), jnp.float32),
        ],
    )(cat, W1, b1c, W2, b2c)


def kernel(fine_ids, coarse_ids, fine_W, coarse_W, W1, b1, W2, b2):
    b, l = fine_ids.shape
    dim = fine_W.shape[1]
    n = b * l
    nw = 32
    nchunks = n // (nw * CHUNK)
    fi = fine_ids.T.astype(jnp.int32).reshape(nw, nchunks, CHUNK)
    ci = coarse_ids.T.astype(jnp.int32).reshape(nw, nchunks, CHUNK)
    cat = _make_sc_gather(n, dim, nw)(fi, ci, fine_W, coarse_W)
    out_p, g_p = _tc_mlp(cat, W1, b1, W2, b2, b, l)
    out = jnp.transpose(out_p, (2, 0, 1))
    g = jnp.transpose(g_p, (2, 0, 1))
    return out, g


# speedup vs baseline: 6.7033x; 1.1121x over previous
import functools

import jax
import jax.numpy as jnp
from jax import lax
from jax.experimental import pallas as pl
from jax.experimental.pallas import tpu as pltpu
from jax.experimental.pallas import tpu_sc as plsc

CHUNK = 128


def _sc_gather_body(nchunks, dim, fi_hbm, ci_hbm, fw_hbm, cw_hbm,
                    cat_hbm, idx_f, idx_c, rows_f, rows_c, sem_f, sem_c):
    nc = 2
    wid = lax.axis_index("s") * nc + lax.axis_index("c")
    base = wid * nchunks * CHUNK

    pltpu.sync_copy(fi_hbm.at[wid], idx_f)
    pltpu.sync_copy(ci_hbm.at[wid], idx_c)

    def body(j, carry):
        r = base + j * CHUNK
        cf = pltpu.async_copy(fw_hbm.at[idx_f.at[j]], rows_f, sem_f)
        cc = pltpu.async_copy(cw_hbm.at[idx_c.at[j]], rows_c, sem_c)
        cf.wait()
        cc.wait()
        pltpu.sync_copy(rows_f, cat_hbm.at[pl.ds(r, CHUNK), pl.ds(0, dim)])
        pltpu.sync_copy(rows_c, cat_hbm.at[pl.ds(r, CHUNK), pl.ds(dim, dim)])
        return carry

    lax.fori_loop(0, nchunks, body, 0)


def _make_sc_gather(n_rows, dim, nw):
    nchunks = n_rows // (nw * CHUNK)
    mesh = plsc.VectorSubcoreMesh(core_axis_name="c", subcore_axis_name="s")
    return functools.partial(
        pl.kernel,
        mesh=mesh,
        compiler_params=pltpu.CompilerParams(use_tc_tiling_on_sc=False),
        out_type=jax.ShapeDtypeStruct((n_rows, 2 * dim), jnp.float32),
        scratch_types=[
            pltpu.VMEM((nchunks, CHUNK), jnp.int32),
            pltpu.VMEM((nchunks, CHUNK), jnp.int32),
            pltpu.VMEM((CHUNK, dim), jnp.float32),
            pltpu.VMEM((CHUNK, dim), jnp.float32),
            pltpu.SemaphoreType.DMA,
            pltpu.SemaphoreType.DMA,
        ],
    )(functools.partial(_sc_gather_body, nchunks, dim))


def _mlp_body(dim, cat_ref, w1_ref, b1_ref, w2_ref, b2_ref, out_ref, g_ref):
    cat_t = cat_ref[...].T
    h = lax.dot_general(w1_ref[...], cat_t, (((0,), (0,)), ((), ())),
                        preferred_element_type=jnp.float32)
    h = jnp.maximum(h + b1_ref[...], 0.0)
    z = jnp.sum(h * w2_ref[...], axis=0, keepdims=True) + b2_ref[...]
    g = jax.nn.sigmoid(z)
    f = cat_t[:dim, :]
    c = cat_t[dim:, :]
    out_ref[0] = c + g * (f - c)
    g_ref[...] = g[None]


def _mlp_body_alias(dim, cat_ref, w1_ref, b1_ref, w2_ref, b2_ref,
                    _prev_out, _prev_g, out_ref, g_ref):
    _mlp_body(dim, cat_ref, w1_ref, b1_ref, w2_ref, b2_ref, out_ref, g_ref)


def _tc_mlp_chunk(cat_c, W1, b1c, W2, b2c, b, l, blk0, prev, rblk=4096):
    n_c, two_dim = cat_c.shape
    dim = two_dim // 2
    hdim = W1.shape[1]
    bpl = b // rblk
    grid = (n_c // rblk,)
    out_map = lambda i: ((blk0 + i) // bpl, 0, (blk0 + i) % bpl)
    gate_map = lambda i: ((blk0 + i) // bpl, 0, (blk0 + i) % bpl)
    in_specs = [
        pl.BlockSpec((rblk, two_dim), lambda i: (i, 0)),
        pl.BlockSpec((two_dim, hdim), lambda i: (0, 0)),
        pl.BlockSpec((hdim, 1), lambda i: (0, 0)),
        pl.BlockSpec((hdim, 1), lambda i: (0, 0)),
        pl.BlockSpec((1, 1), lambda i: (0, 0)),
    ]
    args = [cat_c, W1, b1c, W2, b2c]
    if prev is None:
        body = functools.partial(_mlp_body, dim)
        aliases = {}
    else:
        body = functools.partial(_mlp_body_alias, dim)
        in_specs += [
            pl.BlockSpec((1, 8, 128), lambda i: (0, 0, 0)),
            pl.BlockSpec((1, 1, 128), lambda i: (0, 0, 0)),
        ]
        args += list(prev)
        aliases = {5: 0, 6: 1}
    return pl.pallas_call(
        body,
        grid=grid,
        in_specs=in_specs,
        out_specs=[
            pl.BlockSpec((1, dim, rblk), out_map),
            pl.BlockSpec((1, 1, rblk), gate_map),
        ],
        out_shape=[
            jax.ShapeDtypeStruct((l, dim, b), jnp.float32),
            jax.ShapeDtypeStruct((l, 1, b), jnp.float32),
        ],
        input_output_aliases=aliases,
    )(*args)


K_CHUNKS = 4
RBLK = 4096


def kernel(fine_ids, coarse_ids, fine_W, coarse_W, W1, b1, W2, b2):
    b, l = fine_ids.shape
    dim = fine_W.shape[1]
    n = b * l
    nw = 32
    kc = K_CHUNKS
    n_c = n // kc
    nchunks = n_c // (nw * CHUNK)
    fi = fine_ids.T.astype(jnp.int32).reshape(kc, nw, nchunks, CHUNK)
    ci = coarse_ids.T.astype(jnp.int32).reshape(kc, nw, nchunks, CHUNK)
    b1c = b1.reshape(-1, 1)
    b2c = b2.reshape(1, 1)
    gather = _make_sc_gather(n_c, dim, nw)
    blocks_per_chunk = n_c // RBLK
    prev = None
    for c in range(kc):
        cat_c = gather(fi[c], ci[c], fine_W, coarse_W)
        prev = _tc_mlp_chunk(cat_c, W1, b1c, W2, b2c, b, l,
                             c * blocks_per_chunk, prev, rblk=RBLK)
    out_p, g_p = prev
    out = jnp.transpose(out_p, (2, 0, 1))
    g = jnp.transpose(g_p, (2, 0, 1))
    return out, g


# speedup vs baseline: 6.9280x; 1.0335x over previous
import functools

import jax
import jax.numpy as jnp
from jax import lax
from jax.experimental import pallas as pl
from jax.experimental.pallas import tpu as pltpu
from jax.experimental.pallas import tpu_sc as plsc

CHUNK = 128


NBUF = 4
DELAY = 3


def _sc_gather_body(nchunks, dim, fi_hbm, ci_hbm, fw_hbm, cw_hbm,
                    cat_hbm, idx_f, idx_c, rows_f, rows_c,
                    sem_gf, sem_gc, sem_sf, sem_sc):
    nc = 2
    wid = lax.axis_index("s") * nc + lax.axis_index("c")
    base = wid * nchunks * CHUNK

    pltpu.sync_copy(fi_hbm.at[wid], idx_f)
    pltpu.sync_copy(ci_hbm.at[wid], idx_c)

    gf, gc, sf, sc_ = {}, {}, {}, {}
    for t in range(nchunks + DELAY):
        if t < nchunks:
            b = t % NBUF
            if t >= NBUF:
                sf[t - NBUF].wait()
                sc_[t - NBUF].wait()
            gf[t] = pltpu.async_copy(fw_hbm.at[idx_f.at[t]], rows_f.at[b],
                                     sem_gf)
            gc[t] = pltpu.async_copy(cw_hbm.at[idx_c.at[t]], rows_c.at[b],
                                     sem_gc)
        i = t - DELAY
        if 0 <= i < nchunks:
            gf[i].wait()
            gc[i].wait()
            r = base + i * CHUNK
            sf[i] = pltpu.async_copy(
                rows_f.at[i % NBUF],
                cat_hbm.at[pl.ds(r, CHUNK), pl.ds(0, dim)], sem_sf)
            sc_[i] = pltpu.async_copy(
                rows_c.at[i % NBUF],
                cat_hbm.at[pl.ds(r, CHUNK), pl.ds(dim, dim)], sem_sc)
    for i in range(max(0, nchunks - NBUF), nchunks):
        sf[i].wait()
        sc_[i].wait()


def _make_sc_gather(n_rows, dim, nw):
    nchunks = n_rows // (nw * CHUNK)
    mesh = plsc.VectorSubcoreMesh(core_axis_name="c", subcore_axis_name="s")
    return functools.partial(
        pl.kernel,
        mesh=mesh,
        compiler_params=pltpu.CompilerParams(use_tc_tiling_on_sc=False),
        out_type=jax.ShapeDtypeStruct((n_rows, 2 * dim), jnp.float32),
        scratch_types=[
            pltpu.VMEM((nchunks, CHUNK), jnp.int32),
            pltpu.VMEM((nchunks, CHUNK), jnp.int32),
            pltpu.VMEM((NBUF, CHUNK, dim), jnp.float32),
            pltpu.VMEM((NBUF, CHUNK, dim), jnp.float32),
            pltpu.SemaphoreType.DMA,
            pltpu.SemaphoreType.DMA,
            pltpu.SemaphoreType.DMA,
            pltpu.SemaphoreType.DMA,
        ],
    )(functools.partial(_sc_gather_body, nchunks, dim))


def _mlp_body(dim, cat_ref, w1_ref, b1_ref, w2_ref, b2_ref, out_ref, g_ref):
    cat_t = cat_ref[...].T
    h = lax.dot_general(w1_ref[...], cat_t, (((0,), (0,)), ((), ())),
                        preferred_element_type=jnp.float32)
    h = jnp.maximum(h + b1_ref[...], 0.0)
    z = jnp.sum(h * w2_ref[...], axis=0, keepdims=True) + b2_ref[...]
    g = jax.nn.sigmoid(z)
    f = cat_t[:dim, :]
    c = cat_t[dim:, :]
    out_ref[0] = c + g * (f - c)
    g_ref[...] = g[None]


def _mlp_body_alias(dim, cat_ref, w1_ref, b1_ref, w2_ref, b2_ref,
                    _prev_out, _prev_g, out_ref, g_ref):
    _mlp_body(dim, cat_ref, w1_ref, b1_ref, w2_ref, b2_ref, out_ref, g_ref)


def _tc_mlp_chunk(cat_c, W1, b1c, W2, b2c, b, l, blk0, prev, rblk=4096):
    n_c, two_dim = cat_c.shape
    dim = two_dim // 2
    hdim = W1.shape[1]
    bpl = b // rblk
    grid = (n_c // rblk,)
    out_map = lambda i: ((blk0 + i) // bpl, 0, (blk0 + i) % bpl)
    gate_map = lambda i: ((blk0 + i) // bpl, 0, (blk0 + i) % bpl)
    in_specs = [
        pl.BlockSpec((rblk, two_dim), lambda i: (i, 0)),
        pl.BlockSpec((two_dim, hdim), lambda i: (0, 0)),
        pl.BlockSpec((hdim, 1), lambda i: (0, 0)),
        pl.BlockSpec((hdim, 1), lambda i: (0, 0)),
        pl.BlockSpec((1, 1), lambda i: (0, 0)),
    ]
    args = [cat_c, W1, b1c, W2, b2c]
    if prev is None:
        body = functools.partial(_mlp_body, dim)
        aliases = {}
    else:
        body = functools.partial(_mlp_body_alias, dim)
        in_specs += [
            pl.BlockSpec((1, 8, 128), lambda i: (0, 0, 0)),
            pl.BlockSpec((1, 1, 128), lambda i: (0, 0, 0)),
        ]
        args += list(prev)
        aliases = {5: 0, 6: 1}
    return pl.pallas_call(
        body,
        grid=grid,
        in_specs=in_specs,
        out_specs=[
            pl.BlockSpec((1, dim, rblk), out_map),
            pl.BlockSpec((1, 1, rblk), gate_map),
        ],
        out_shape=[
            jax.ShapeDtypeStruct((l, dim, b), jnp.float32),
            jax.ShapeDtypeStruct((l, 1, b), jnp.float32),
        ],
        input_output_aliases=aliases,
    )(*args)


K_CHUNKS = 4
RBLK = 4096


def kernel(fine_ids, coarse_ids, fine_W, coarse_W, W1, b1, W2, b2):
    b, l = fine_ids.shape
    dim = fine_W.shape[1]
    n = b * l
    nw = 32
    kc = K_CHUNKS
    n_c = n // kc
    nchunks = n_c // (nw * CHUNK)
    fi = fine_ids.T.astype(jnp.int32).reshape(kc, nw, nchunks, CHUNK)
    ci = coarse_ids.T.astype(jnp.int32).reshape(kc, nw, nchunks, CHUNK)
    b1c = b1.reshape(-1, 1)
    b2c = b2.reshape(1, 1)
    gather = _make_sc_gather(n_c, dim, nw)
    blocks_per_chunk = n_c // RBLK
    prev = None
    for c in range(kc):
        cat_c = gather(fi[c], ci[c], fine_W, coarse_W)
        prev = _tc_mlp_chunk(cat_c, W1, b1c, W2, b2c, b, l,
                             c * blocks_per_chunk, prev, rblk=RBLK)
    out_p, g_p = prev
    out = jnp.transpose(out_p, (2, 0, 1))
    g = jnp.transpose(g_p, (2, 0, 1))
    return out, g


# speedup vs baseline: 7.0682x; 1.0202x over previous
import functools

import jax
import jax.numpy as jnp
from jax import lax
from jax.experimental import pallas as pl
from jax.experimental.pallas import tpu as pltpu
from jax.experimental.pallas import tpu_sc as plsc

CHUNK = 128


NBUF = 4
DELAY = 3


def _sc_gather_body(nchunks, dim, fi_hbm, ci_hbm, fw_hbm, cw_hbm,
                    cat_hbm, idx_f, idx_c, rows_f, rows_c,
                    sem_gf, sem_gc, sem_sf, sem_sc):
    nc = 2
    wid = lax.axis_index("s") * nc + lax.axis_index("c")
    base = wid * nchunks * CHUNK

    pltpu.sync_copy(fi_hbm.at[wid], idx_f)
    pltpu.sync_copy(ci_hbm.at[wid], idx_c)

    gf, gc, sf, sc_ = {}, {}, {}, {}
    for t in range(nchunks + DELAY):
        if t < nchunks:
            b = t % NBUF
            if t >= NBUF:
                sf[t - NBUF].wait()
                sc_[t - NBUF].wait()
            gf[t] = pltpu.async_copy(fw_hbm.at[idx_f.at[t]], rows_f.at[b],
                                     sem_gf)
            gc[t] = pltpu.async_copy(cw_hbm.at[idx_c.at[t]], rows_c.at[b],
                                     sem_gc)
        i = t - DELAY
        if 0 <= i < nchunks:
            gf[i].wait()
            gc[i].wait()
            r = base + i * CHUNK
            sf[i] = pltpu.async_copy(
                rows_f.at[i % NBUF],
                cat_hbm.at[pl.ds(r, CHUNK), pl.ds(0, dim)], sem_sf)
            sc_[i] = pltpu.async_copy(
                rows_c.at[i % NBUF],
                cat_hbm.at[pl.ds(r, CHUNK), pl.ds(dim, dim)], sem_sc)
    for i in range(max(0, nchunks - NBUF), nchunks):
        sf[i].wait()
        sc_[i].wait()


def _make_sc_gather(n_rows, dim, nw):
    nchunks = n_rows // (nw * CHUNK)
    mesh = plsc.VectorSubcoreMesh(core_axis_name="c", subcore_axis_name="s")
    return functools.partial(
        pl.kernel,
        mesh=mesh,
        compiler_params=pltpu.CompilerParams(use_tc_tiling_on_sc=False),
        out_type=jax.ShapeDtypeStruct((n_rows, 2 * dim), jnp.float32),
        scratch_types=[
            pltpu.VMEM((nchunks, CHUNK), jnp.int32),
            pltpu.VMEM((nchunks, CHUNK), jnp.int32),
            pltpu.VMEM((NBUF, CHUNK, dim), jnp.float32),
            pltpu.VMEM((NBUF, CHUNK, dim), jnp.float32),
            pltpu.SemaphoreType.DMA,
            pltpu.SemaphoreType.DMA,
            pltpu.SemaphoreType.DMA,
            pltpu.SemaphoreType.DMA,
        ],
    )(functools.partial(_sc_gather_body, nchunks, dim))


def _mlp_body(dim, cat_ref, w1_ref, b1_ref, w2_ref, b2_ref, out_ref, g_ref):
    cat_t = cat_ref[...].T
    h = lax.dot_general(w1_ref[...], cat_t, (((0,), (0,)), ((), ())),
                        preferred_element_type=jnp.float32)
    h = jnp.maximum(h + b1_ref[...], 0.0)
    z = jnp.sum(h * w2_ref[...], axis=0, keepdims=True) + b2_ref[...]
    g = jax.nn.sigmoid(z)
    f = cat_t[:dim, :]
    c = cat_t[dim:, :]
    out_ref[0] = c + g * (f - c)
    g_ref[...] = g[None]


def _mlp_body_alias(dim, cat_ref, w1_ref, b1_ref, w2_ref, b2_ref,
                    _prev_out, _prev_g, out_ref, g_ref):
    _mlp_body(dim, cat_ref, w1_ref, b1_ref, w2_ref, b2_ref, out_ref, g_ref)


def _tc_mlp_chunk(cat_c, W1, b1c, W2, b2c, b, l, blk0, prev, rblk=4096):
    n_c, two_dim = cat_c.shape
    dim = two_dim // 2
    hdim = W1.shape[1]
    bpl = b // rblk
    grid = (n_c // rblk,)
    out_map = lambda i: ((blk0 + i) // bpl, 0, (blk0 + i) % bpl)
    gate_map = lambda i: ((blk0 + i) // bpl, 0, (blk0 + i) % bpl)
    in_specs = [
        pl.BlockSpec((rblk, two_dim), lambda i: (i, 0)),
        pl.BlockSpec((two_dim, hdim), lambda i: (0, 0)),
        pl.BlockSpec((hdim, 1), lambda i: (0, 0)),
        pl.BlockSpec((hdim, 1), lambda i: (0, 0)),
        pl.BlockSpec((1, 1), lambda i: (0, 0)),
    ]
    args = [cat_c, W1, b1c, W2, b2c]
    if prev is None:
        body = functools.partial(_mlp_body, dim)
        aliases = {}
    else:
        body = functools.partial(_mlp_body_alias, dim)
        in_specs += [
            pl.BlockSpec((1, 8, 128), lambda i: (0, 0, 0)),
            pl.BlockSpec((1, 1, 128), lambda i: (0, 0, 0)),
        ]
        args += list(prev)
        aliases = {5: 0, 6: 1}
    return pl.pallas_call(
        body,
        grid=grid,
        in_specs=in_specs,
        out_specs=[
            pl.BlockSpec((1, dim, rblk), out_map),
            pl.BlockSpec((1, 1, rblk), gate_map),
        ],
        out_shape=[
            jax.ShapeDtypeStruct((l, dim, b), jnp.float32),
            jax.ShapeDtypeStruct((l, 1, b), jnp.float32),
        ],
        input_output_aliases=aliases,
    )(*args)


K_CHUNKS = 4
RBLK = 8192


def kernel(fine_ids, coarse_ids, fine_W, coarse_W, W1, b1, W2, b2):
    b, l = fine_ids.shape
    dim = fine_W.shape[1]
    n = b * l
    nw = 32
    kc = K_CHUNKS
    n_c = n // kc
    nchunks = n_c // (nw * CHUNK)
    fi = fine_ids.T.astype(jnp.int32).reshape(kc, nw, nchunks, CHUNK)
    ci = coarse_ids.T.astype(jnp.int32).reshape(kc, nw, nchunks, CHUNK)
    b1c = b1.reshape(-1, 1)
    b2c = b2.reshape(1, 1)
    gather = _make_sc_gather(n_c, dim, nw)
    blocks_per_chunk = n_c // RBLK
    prev = None
    for c in range(kc):
        cat_c = gather(fi[c], ci[c], fine_W, coarse_W)
        prev = _tc_mlp_chunk(cat_c, W1, b1c, W2, b2c, b, l,
                             c * blocks_per_chunk, prev, rblk=RBLK)
    out_p, g_p = prev
    out = jnp.transpose(out_p, (2, 0, 1))
    g = jnp.transpose(g_p, (2, 0, 1))
    return out, g
